# baseline (speedup 1.0000x reference)
import jax
import jax.numpy as jnp
from jax import lax
from jax.experimental import pallas as pl
from jax.experimental.pallas import tpu as pltpu
from jax.experimental.pallas import tpu_sc as plsc

B, S, D = 8, 4096, 256
H, W = 64, 64
HW = H * W
LAT_MIN, LAT_MAX = -18.414806, -7.9918404
LON_MIN, LON_MAX = 21.167515, 35.316326
MEAN_LAT, SCALE_LAT = -13.1573589, 2.86575632
MEAN_LON, SCALE_LON = 27.7910228, 3.56468299

NC, NS = 2, 16
TPS = S // NS
TPC = 32
NCH = TPS // TPC
LANES = 16
DH = D // NC


def _prep_body(xa_ref, xb_ref, xv_ref, lin_ref, ew_ref):
    xa = xa_ref[...]
    xb = xb_ref[...]
    xv = xv_ref[...]
    lat01 = (xa * SCALE_LAT + (MEAN_LAT - LAT_MIN)) * (1.0 / (LAT_MAX - LAT_MIN))
    lon01 = (xb * SCALE_LON + (MEAN_LON - LON_MIN)) * (1.0 / (LON_MAX - LON_MIN))
    lat01 = 1.0 - jnp.clip(lat01, 0.0, 1.0)
    lon01 = jnp.clip(lon01, 0.0, 1.0)
    valid = (xv > 0.0).astype(jnp.float32)
    iyf = lat01 * (H - 1)
    ixf = lon01 * (W - 1)
    iy0 = jnp.clip(jnp.floor(iyf).astype(jnp.int32), 0, H - 1)
    ix0 = jnp.clip(jnp.floor(ixf).astype(jnp.int32), 0, W - 1)
    iy1 = jnp.minimum(iy0 + 1, H - 1)
    ix1 = jnp.minimum(ix0 + 1, W - 1)
    wy = iyf - iy0.astype(jnp.float32)
    wx = ixf - ix0.astype(jnp.float32)
    lin_ref[0] = iy0 * W + ix0
    lin_ref[1] = iy0 * W + ix1
    lin_ref[2] = iy1 * W + ix0
    lin_ref[3] = iy1 * W + ix1
    ew_ref[0] = (1.0 - wy) * (1.0 - wx) * valid
    ew_ref[1] = (1.0 - wy) * wx * valid
    ew_ref[2] = wy * (1.0 - wx) * valid
    ew_ref[3] = wy * wx * valid


_prep = pl.pallas_call(
    _prep_body,
    out_shape=[
        jax.ShapeDtypeStruct((4, B, S), jnp.int32),
        jax.ShapeDtypeStruct((4, B, S), jnp.float32),
    ],
)


def _cnts_body(xa_ref, xb_ref, xv_ref, cnt_ref):
    xa = xa_ref[0, 0]
    xb = xb_ref[0, 0]
    xv = xv_ref[0, 0]
    lat01 = (xa * SCALE_LAT + (MEAN_LAT - LAT_MIN)) * (1.0 / (LAT_MAX - LAT_MIN))
    lon01 = (xb * SCALE_LON + (MEAN_LON - LON_MIN)) * (1.0 / (LON_MAX - LON_MIN))
    lat01 = 1.0 - jnp.clip(lat01, 0.0, 1.0)
    lon01 = jnp.clip(lon01, 0.0, 1.0)
    valid = (xv > 0.0).astype(jnp.float32)
    iyf = lat01 * (H - 1)
    ixf = lon01 * (W - 1)
    cols = lax.broadcasted_iota(jnp.int32, (S, H), 1).astype(jnp.float32)
    av = (jnp.maximum(0.0, 1.0 - jnp.abs(iyf[:, None] - cols))
          * valid[:, None])
    bx = jnp.maximum(0.0, 1.0 - jnp.abs(ixf[:, None] - cols))
    cnt_ref[0] = lax.dot_general(av, bx, (((0,), (0,)), ((), ())),
                                 preferred_element_type=jnp.float32,
                                 precision=lax.Precision.HIGHEST)


_cnts = pl.pallas_call(
    _cnts_body,
    grid=(B,),
    in_specs=[
        pl.BlockSpec((1, 1, S), lambda b: (b, 0, 0)),
        pl.BlockSpec((1, 1, S), lambda b: (b, 0, 0)),
        pl.BlockSpec((1, 1, S), lambda b: (b, 0, 0)),
    ],
    out_specs=pl.BlockSpec((1, H, W), lambda b: (b, 0, 0)),
    out_shape=jax.ShapeDtypeStruct((B, H, W), jnp.float32),
)


def _sc_body(feats_hbm, lin_hbm, ew_hbm, out_hbm,
             grid_sp, fbufs, wbufs, zbuf, idx_ref, ewb,
             fsems, ssems, wsem):
    c = lax.axis_index("c")
    sid = lax.axis_index("s")
    base = pl.multiple_of(sid * TPS, TPS)
    dlo = pl.multiple_of(c * DH, DH)
    myrows = pl.ds(base, TPS)

    zv = jnp.zeros((LANES,), jnp.float32)

    @pl.loop(0, TPS)
    def _zero_tpl(r):
        for j in range(DH // LANES):
            zbuf[r, pl.ds(j * LANES, LANES)] = zv

    def _feats_dma(b, ch, buf):
        src = feats_hbm.at[b, pl.ds(pl.multiple_of(base + ch * TPC, TPC), TPC),
                           pl.ds(dlo, DH)]
        return pltpu.async_copy(src, fbufs.at[buf], fsems.at[buf])

    @pl.loop(0, B)
    def _round(b):
        pltpu.sync_copy(lin_hbm.at[b, pl.ds(pl.multiple_of(sid * NCH, NCH),
                                            NCH)], idx_ref)
        pltpu.sync_copy(ew_hbm.at[b, sid], ewb)
        f0 = _feats_dma(b, 0, 0)
        @pl.when(b > 0)
        def _():
            pltpu.make_async_copy(grid_sp.at[myrows],
                                  out_hbm.at[b, myrows, pl.ds(dlo, DH)],
                                  wsem).wait()
        pltpu.sync_copy(zbuf, grid_sp.at[myrows])
        plsc.subcore_barrier()
        f0.wait()
        nxt = _feats_dma(b, 1, 1)
        for ch in range(NCH):
            bsel = ch % 2
            if ch >= 2:
                pltpu.make_async_copy(wbufs.at[bsel],
                                      grid_sp.at[idx_ref.at[ch - 2]],
                                      ssems.at[bsel]).wait()
            if ch >= 1:
                nxt.wait()
                if ch < NCH - 1:
                    nxt = _feats_dma(b, ch + 1, (ch + 1) % 2)

            @plsc.parallel_loop(0, TPC, unroll=4)
            def _tok(t):
                fr = [fbufs[bsel, t, pl.ds(j * LANES, LANES)]
                      for j in range(DH // LANES)]
                lvec = jnp.full((LANES,), t % LANES, jnp.int32)
                for k in range(4):
                    wrow = ewb[k, ch * (TPC // LANES) + t // LANES]
                    wv = lax.gather(
                        wrow, lvec[:, None],
                        lax.GatherDimensionNumbers(
                            offset_dims=(), collapsed_slice_dims=(0,),
                            start_index_map=(0,)),
                        slice_sizes=(1,),
                        mode=lax.GatherScatterMode.PROMISE_IN_BOUNDS)
                    for j in range(DH // LANES):
                        wbufs[bsel, k * TPC + t, pl.ds(j * LANES, LANES)] = \
                            fr[j] * wv
            pltpu.async_copy(wbufs.at[bsel], grid_sp.at[idx_ref.at[ch]],
                             ssems.at[bsel], add=True)
        for ch in (NCH - 2, NCH - 1):
            pltpu.make_async_copy(wbufs.at[ch % 2],
                                  grid_sp.at[idx_ref.at[ch]],
                                  ssems.at[ch % 2]).wait()
        plsc.subcore_barrier()
        pltpu.async_copy(grid_sp.at[myrows],
                         out_hbm.at[b, myrows, pl.ds(dlo, DH)], wsem)

    pltpu.make_async_copy(grid_sp.at[myrows],
                          out_hbm.at[B - 1, myrows, pl.ds(dlo, DH)],
                          wsem).wait()


def _sc_scatter(feats, lin2, ew):
    mesh = plsc.VectorSubcoreMesh(core_axis_name="c", subcore_axis_name="s")
    return pl.kernel(
        _sc_body,
        out_type=jax.ShapeDtypeStruct((B, HW, D), jnp.float32),
        mesh=mesh,
        scratch_types=[
            pltpu.VMEM_SHARED((HW, DH), jnp.float32),
            pltpu.VMEM((2, TPC, DH), jnp.float32),
            pltpu.VMEM((2, 4 * TPC, DH), jnp.float32),
            pltpu.VMEM((TPS, DH), jnp.float32),
            pltpu.VMEM((NCH, 4 * TPC), jnp.int32),
            pltpu.VMEM((4, TPS // LANES, LANES), jnp.float32),
            pltpu.SemaphoreType.DMA((2,)),
            pltpu.SemaphoreType.DMA((2,)),
            pltpu.SemaphoreType.DMA,
        ],
    )(feats, lin2, ew)


def _post_body(g_ref, c_ref, o_ref):
    g = g_ref[0]
    cd = jnp.maximum(c_ref[0, 0], 1.0)
    o_ref[0] = g.T / cd[None, :]


_post = pl.pallas_call(
    _post_body,
    grid=(B,),
    in_specs=[
        pl.BlockSpec((1, HW, D), lambda b: (b, 0, 0)),
        pl.BlockSpec((1, 1, HW), lambda b: (b, 0, 0)),
    ],
    out_specs=pl.BlockSpec((1, D, HW), lambda b: (b, 0, 0)),
    out_shape=jax.ShapeDtypeStruct((B, D, HW), jnp.float32),
)


def kernel(token_feats, x_raw):
    xa = x_raw[:, :, 0]
    xb = x_raw[:, :, 1]
    xv = x_raw[:, :, 8]
    lin, ew = _prep(xa, xb, xv)
    cnt = _cnts(xa.reshape(B, 1, S), xb.reshape(B, 1, S), xv.reshape(B, 1, S))
    lin2 = (lin.reshape(4, B, NS * NCH, TPC).transpose(1, 2, 0, 3)
            .reshape(B, NS * NCH, 4 * TPC))
    ew3 = (ew.reshape(4, B, NS, TPS // LANES, LANES)
           .transpose(1, 2, 0, 3, 4)
           .reshape(B, NS, 4, TPS // LANES, LANES))
    gridacc = _sc_scatter(token_feats, lin2, ew3)
    out = _post(gridacc, cnt.reshape(B, 1, HW))
    return out.reshape(B, D, H, W)


# speedup vs baseline: 1.0291x; 1.0291x over previous
import jax
import jax.numpy as jnp
from jax import lax
from jax.experimental import pallas as pl
from jax.experimental.pallas import tpu as pltpu
from jax.experimental.pallas import tpu_sc as plsc

B, S, D = 8, 4096, 256
H, W = 64, 64
HW = H * W
LAT_MIN, LAT_MAX = -18.414806, -7.9918404
LON_MIN, LON_MAX = 21.167515, 35.316326
MEAN_LAT, SCALE_LAT = -13.1573589, 2.86575632
MEAN_LON, SCALE_LON = 27.7910228, 3.56468299

NC, NS = 2, 16
TPS = S // NS
TPC = 32
NCH = TPS // TPC
LANES = 16
DH = D // NC


def _prep_body(xa_ref, xb_ref, xv_ref, lin_ref, ew_ref):
    xa = xa_ref[...]
    xb = xb_ref[...]
    xv = xv_ref[...]
    lat01 = (xa * SCALE_LAT + (MEAN_LAT - LAT_MIN)) * (1.0 / (LAT_MAX - LAT_MIN))
    lon01 = (xb * SCALE_LON + (MEAN_LON - LON_MIN)) * (1.0 / (LON_MAX - LON_MIN))
    lat01 = 1.0 - jnp.clip(lat01, 0.0, 1.0)
    lon01 = jnp.clip(lon01, 0.0, 1.0)
    valid = (xv > 0.0).astype(jnp.float32)
    iyf = lat01 * (H - 1)
    ixf = lon01 * (W - 1)
    iy0 = jnp.clip(jnp.floor(iyf).astype(jnp.int32), 0, H - 1)
    ix0 = jnp.clip(jnp.floor(ixf).astype(jnp.int32), 0, W - 1)
    iy1 = jnp.minimum(iy0 + 1, H - 1)
    ix1 = jnp.minimum(ix0 + 1, W - 1)
    wy = iyf - iy0.astype(jnp.float32)
    wx = ixf - ix0.astype(jnp.float32)
    lin_ref[0] = iy0 * W + ix0
    lin_ref[1] = iy0 * W + ix1
    lin_ref[2] = iy1 * W + ix0
    lin_ref[3] = iy1 * W + ix1
    ew_ref[0] = (1.0 - wy) * (1.0 - wx) * valid
    ew_ref[1] = (1.0 - wy) * wx * valid
    ew_ref[2] = wy * (1.0 - wx) * valid
    ew_ref[3] = wy * wx * valid


_prep = pl.pallas_call(
    _prep_body,
    out_shape=[
        jax.ShapeDtypeStruct((4, B, S), jnp.int32),
        jax.ShapeDtypeStruct((4, B, S), jnp.float32),
    ],
)


def _cnts_body(xa_ref, xb_ref, xv_ref, cnt_ref):
    xa = xa_ref[0, 0]
    xb = xb_ref[0, 0]
    xv = xv_ref[0, 0]
    lat01 = (xa * SCALE_LAT + (MEAN_LAT - LAT_MIN)) * (1.0 / (LAT_MAX - LAT_MIN))
    lon01 = (xb * SCALE_LON + (MEAN_LON - LON_MIN)) * (1.0 / (LON_MAX - LON_MIN))
    lat01 = 1.0 - jnp.clip(lat01, 0.0, 1.0)
    lon01 = jnp.clip(lon01, 0.0, 1.0)
    valid = (xv > 0.0).astype(jnp.float32)
    iyf = lat01 * (H - 1)
    ixf = lon01 * (W - 1)
    cols = lax.broadcasted_iota(jnp.int32, (S, H), 1).astype(jnp.float32)
    av = (jnp.maximum(0.0, 1.0 - jnp.abs(iyf[:, None] - cols))
          * valid[:, None])
    bx = jnp.maximum(0.0, 1.0 - jnp.abs(ixf[:, None] - cols))
    cnt_ref[0] = lax.dot_general(av, bx, (((0,), (0,)), ((), ())),
                                 preferred_element_type=jnp.float32,
                                 precision=lax.Precision.HIGHEST)


_cnts = pl.pallas_call(
    _cnts_body,
    grid=(B,),
    in_specs=[
        pl.BlockSpec((1, 1, S), lambda b: (b, 0, 0)),
        pl.BlockSpec((1, 1, S), lambda b: (b, 0, 0)),
        pl.BlockSpec((1, 1, S), lambda b: (b, 0, 0)),
    ],
    out_specs=pl.BlockSpec((1, H, W), lambda b: (b, 0, 0)),
    out_shape=jax.ShapeDtypeStruct((B, H, W), jnp.float32),
)


def _sc_body(feats_hbm, lin_hbm, ew_hbm, out_hbm,
             grid_sp, fbufs, wbufs, zbuf, idx_ref, ewb,
             fsems, ssems, wsem, stsem):
    c = lax.axis_index("c")
    sid = lax.axis_index("s")
    base = pl.multiple_of(sid * TPS, TPS)
    dlo = pl.multiple_of(c * DH, DH)
    myrows = pl.ds(base, TPS)

    zv = jnp.zeros((LANES,), jnp.float32)

    @pl.loop(0, TPS)
    def _zero_tpl(r):
        for j in range(DH // LANES):
            zbuf[r, pl.ds(j * LANES, LANES)] = zv

    def _feats_dma(b, ch, buf):
        src = feats_hbm.at[b, pl.ds(pl.multiple_of(base + ch * TPC, TPC), TPC),
                           pl.ds(dlo, DH)]
        return pltpu.async_copy(src, fbufs.at[buf], fsems.at[buf])

    @pl.loop(0, B)
    def _round(b):
        s0 = pltpu.async_copy(
            lin_hbm.at[b, pl.ds(pl.multiple_of(sid * NCH, NCH), NCH)],
            idx_ref, stsem)
        s1 = pltpu.async_copy(ew_hbm.at[b, sid], ewb, stsem)
        f0 = _feats_dma(b, 0, 0)
        @pl.when(b > 0)
        def _():
            pltpu.make_async_copy(grid_sp.at[myrows],
                                  out_hbm.at[b, myrows, pl.ds(dlo, DH)],
                                  wsem).wait()
        pltpu.sync_copy(zbuf, grid_sp.at[myrows])
        plsc.subcore_barrier()
        s0.wait()
        s1.wait()
        f0.wait()
        nxt = _feats_dma(b, 1, 1)
        for ch in range(NCH):
            bsel = ch % 2
            if ch >= 2:
                pltpu.make_async_copy(wbufs.at[bsel],
                                      grid_sp.at[idx_ref.at[ch - 2]],
                                      ssems.at[bsel]).wait()
            if ch >= 1:
                nxt.wait()
                if ch < NCH - 1:
                    nxt = _feats_dma(b, ch + 1, (ch + 1) % 2)

            @plsc.parallel_loop(0, TPC, unroll=2)
            def _tok(t):
                fr = [fbufs[bsel, t, pl.ds(j * LANES, LANES)]
                      for j in range(DH // LANES)]
                lvec = jnp.full((LANES,), t % LANES, jnp.int32)
                for k in range(4):
                    wrow = ewb[k, ch * (TPC // LANES) + t // LANES]
                    wv = lax.gather(
                        wrow, lvec[:, None],
                        lax.GatherDimensionNumbers(
                            offset_dims=(), collapsed_slice_dims=(0,),
                            start_index_map=(0,)),
                        slice_sizes=(1,),
                        mode=lax.GatherScatterMode.PROMISE_IN_BOUNDS)
                    for j in range(DH // LANES):
                        wbufs[bsel, k * TPC + t, pl.ds(j * LANES, LANES)] = \
                            fr[j] * wv
            pltpu.async_copy(wbufs.at[bsel], grid_sp.at[idx_ref.at[ch]],
                             ssems.at[bsel], add=True)
        for ch in (NCH - 2, NCH - 1):
            pltpu.make_async_copy(wbufs.at[ch % 2],
                                  grid_sp.at[idx_ref.at[ch]],
                                  ssems.at[ch % 2]).wait()
        plsc.subcore_barrier()
        pltpu.async_copy(grid_sp.at[myrows],
                         out_hbm.at[b, myrows, pl.ds(dlo, DH)], wsem)

    pltpu.make_async_copy(grid_sp.at[myrows],
                          out_hbm.at[B - 1, myrows, pl.ds(dlo, DH)],
                          wsem).wait()


def _sc_scatter(feats, lin2, ew):
    mesh = plsc.VectorSubcoreMesh(core_axis_name="c", subcore_axis_name="s")
    return pl.kernel(
        _sc_body,
        out_type=jax.ShapeDtypeStruct((B, HW, D), jnp.float32),
        mesh=mesh,
        scratch_types=[
            pltpu.VMEM_SHARED((HW, DH), jnp.float32),
            pltpu.VMEM((2, TPC, DH), jnp.float32),
            pltpu.VMEM((2, 4 * TPC, DH), jnp.float32),
            pltpu.VMEM((TPS, DH), jnp.float32),
            pltpu.VMEM((NCH, 4 * TPC), jnp.int32),
            pltpu.VMEM((4, TPS // LANES, LANES), jnp.float32),
            pltpu.SemaphoreType.DMA((2,)),
            pltpu.SemaphoreType.DMA((2,)),
            pltpu.SemaphoreType.DMA,
            pltpu.SemaphoreType.DMA,
        ],
    )(feats, lin2, ew)


def _post_body(g_ref, c_ref, o_ref):
    g = g_ref[0]
    cd = jnp.maximum(c_ref[0, 0], 1.0)
    o_ref[0] = g.T / cd[None, :]


_post = pl.pallas_call(
    _post_body,
    grid=(B,),
    in_specs=[
        pl.BlockSpec((1, HW, D), lambda b: (b, 0, 0)),
        pl.BlockSpec((1, 1, HW), lambda b: (b, 0, 0)),
    ],
    out_specs=pl.BlockSpec((1, D, HW), lambda b: (b, 0, 0)),
    out_shape=jax.ShapeDtypeStruct((B, D, HW), jnp.float32),
)


def kernel(token_feats, x_raw):
    xa = x_raw[:, :, 0]
    xb = x_raw[:, :, 1]
    xv = x_raw[:, :, 8]
    lin, ew = _prep(xa, xb, xv)
    cnt = _cnts(xa.reshape(B, 1, S), xb.reshape(B, 1, S), xv.reshape(B, 1, S))
    lin2 = (lin.reshape(4, B, NS * NCH, TPC).transpose(1, 2, 0, 3)
            .reshape(B, NS * NCH, 4 * TPC))
    ew3 = (ew.reshape(4, B, NS, TPS // LANES, LANES)
           .transpose(1, 2, 0, 3, 4)
           .reshape(B, NS, 4, TPS // LANES, LANES))
    gridacc = _sc_scatter(token_feats, lin2, ew3)
    out = _post(gridacc, cnt.reshape(B, 1, HW))
    return out.reshape(B, D, H, W)
